# baseline (speedup 1.0000x reference)
import functools

import jax
import jax.numpy as jnp
from jax import lax
from jax.experimental import pallas as pl
from jax.experimental.pallas import tpu as pltpu
from jax.experimental.pallas import tpu_sc as plsc

N_NODES = 10000
NUM_TX = 6000
E_EDGES = 160000
R_REL = 8
B_BAS = 4
D = 128
NSLOT = R_REL + 1
EPS = 1e-5

NC = 2
NS = 16
LANES = 16
NW = NC * NS

EB = 128
E_PAD = 163840
EPW = E_PAD // NW
NBLK = EPW // EB
BLK_C0 = 80
BLK_C1 = 2 * NBLK - BLK_C0
BLK_MAX = max(BLK_C0, BLK_C1)
ACC_ROWS = 10112
STRIPE = ACC_ROWS // NS
_CHUNKS = []
_off = 0
while _off < STRIPE:
    _CHUNKS.append((_off, min(EB, STRIPE - _off)))
    _off += EB

ROW_BLK = 1000
N_GRID = N_NODES // ROW_BLK


def _fill_rows(rows, val16):
    def _f(i, carry):
        for j in range(D // LANES):
            rows[i, pl.ds(j * LANES, LANES)] = val16
        return carry
    lax.fori_loop(0, EB, _f, 0)


def _zero_acc_stripe(rows, acc, r0):
    _fill_rows(rows, jnp.zeros((LANES,), jnp.float32))
    for off, sz in _CHUNKS:
        pltpu.sync_copy(rows.at[pl.ds(0, sz)], acc.at[pl.ds(r0 + off, sz)])


def _writeback_stripe(acc, rows, part_out, c, r0):
    for off, sz in _CHUNKS:
        r = r0 + off
        pltpu.sync_copy(acc.at[pl.ds(r, sz)], rows.at[pl.ds(0, sz)])
        pltpu.sync_copy(rows.at[pl.ds(0, sz)], part_out.at[c, pl.ds(r, sz)])


def _sc_agg_body(z_hbm, idx_hbm, dst_hbm, part_out,
                 idx2d, dst2d, rows0, acc, semA):
    c = lax.axis_index("c")
    s = lax.axis_index("s")
    r0 = s * STRIPE

    @pl.when(c == 0)
    def _():
        pltpu.sync_copy(idx_hbm.at[pl.ds(s * BLK_C0, BLK_C0)],
                        idx2d.at[pl.ds(0, BLK_C0)])
        pltpu.sync_copy(dst_hbm.at[pl.ds(s * BLK_C0, BLK_C0)],
                        dst2d.at[pl.ds(0, BLK_C0)])

    if BLK_C1 > 0:
        @pl.when(c == 1)
        def _():
            pltpu.sync_copy(
                idx_hbm.at[pl.ds(NS * BLK_C0 + s * BLK_C1, BLK_C1)],
                idx2d.at[pl.ds(0, BLK_C1)])
            pltpu.sync_copy(
                dst_hbm.at[pl.ds(NS * BLK_C0 + s * BLK_C1, BLK_C1)],
                dst2d.at[pl.ds(0, BLK_C1)])

    nb = jnp.where(c == 0, BLK_C0, BLK_C1)
    _zero_acc_stripe(rows0, acc, r0)
    plsc.subcore_barrier()

    @pl.loop(0, nb)
    def _blk(b):
        pltpu.async_copy(z_hbm.at[idx2d.at[b]], rows0, semA).wait()
        pltpu.sync_copy(rows0, acc.at[dst2d.at[b]], add=True)

    plsc.subcore_barrier()
    _writeback_stripe(acc, rows0, part_out, c, r0)


def _sc_count_body(dst_hbm, part_out, dst2d, rows, acc, sem):
    c = lax.axis_index("c")
    s = lax.axis_index("s")
    wid = c * NS + s
    r0 = s * STRIPE
    pltpu.sync_copy(dst_hbm.at[pl.ds(wid * NBLK, NBLK)], dst2d)
    _zero_acc_stripe(rows, acc, r0)
    _fill_rows(rows, jnp.ones((LANES,), jnp.float32))
    plsc.subcore_barrier()

    descs = []
    for b in range(NBLK):
        descs.append(pltpu.async_copy(rows, acc.at[dst2d.at[b]], sem,
                                      add=True))
    for d in descs:
        d.wait()

    plsc.subcore_barrier()
    _writeback_stripe(acc, rows, part_out, c, r0)


@functools.cache
def _sc_agg_kernel():
    return pl.kernel(
        _sc_agg_body,
        out_type=[jax.ShapeDtypeStruct((NC, ACC_ROWS, D), jnp.float32)],
        mesh=plsc.VectorSubcoreMesh(core_axis_name="c", subcore_axis_name="s"),
        scratch_types=[
            pltpu.VMEM((BLK_MAX, EB), jnp.int32),
            pltpu.VMEM((BLK_MAX, EB), jnp.int32),
            pltpu.VMEM((EB, D), jnp.float32),
            pltpu.VMEM_SHARED((ACC_ROWS, D), jnp.float32),
            pltpu.SemaphoreType.DMA,
        ],
        name="sc_rgcn_agg",
    )


@functools.cache
def _sc_count_kernel():
    return pl.kernel(
        _sc_count_body,
        out_type=[jax.ShapeDtypeStruct((NC, ACC_ROWS, D), jnp.float32)],
        mesh=plsc.VectorSubcoreMesh(core_axis_name="c", subcore_axis_name="s"),
        scratch_types=[
            pltpu.VMEM((NBLK, EB), jnp.int32),
            pltpu.VMEM((EB, D), jnp.float32),
            pltpu.VMEM_SHARED((ACC_ROWS, D), jnp.float32),
            pltpu.SemaphoreType.DMA,
        ],
        name="sc_rgcn_count",
    )


def _sc_agg(ztab, eidx, dst):
    return _sc_agg_kernel()(ztab, eidx, dst)[0]


def _sc_count(dst):
    return _sc_count_kernel()(dst)[0]


def _eidx_body(s_ref, t_ref, o_ref):
    o_ref[...] = s_ref[...] * NSLOT + t_ref[...]


def _eidx(src, et):
    return pl.pallas_call(
        _eidx_body,
        out_shape=jax.ShapeDtypeStruct((E_PAD // EB, EB), jnp.int32),
        name="edge_gather_idx",
    )(src, et)


def _wcat_body(comp_ref, basis_ref, root_ref, out_ref):
    for r in range(R_REL):
        acc = comp_ref[r, 0] * basis_ref[0]
        for b in range(1, B_BAS):
            acc = acc + comp_ref[r, b] * basis_ref[b]
        out_ref[:, r * D:(r + 1) * D] = acc
    out_ref[:, R_REL * D:] = root_ref[...]


def _wcat(comp, basis, root):
    return pl.pallas_call(
        _wcat_body,
        out_shape=jax.ShapeDtypeStruct((D, NSLOT * D), jnp.float32),
        in_specs=[
            pl.BlockSpec(memory_space=pltpu.SMEM),
            pl.BlockSpec((B_BAS, D, D), lambda: (0, 0, 0)),
            pl.BlockSpec((D, D), lambda: (0, 0)),
        ],
        out_specs=pl.BlockSpec((D, NSLOT * D), lambda: (0, 0)),
        name="wcat",
    )(comp, basis, root)


def _mm_body(x_ref, w_ref, o_ref):
    o_ref[...] = jnp.dot(x_ref[...], w_ref[...],
                         preferred_element_type=jnp.float32)


def _mm(x, w):
    return pl.pallas_call(
        _mm_body,
        grid=(N_GRID,),
        out_shape=jax.ShapeDtypeStruct((N_NODES, NSLOT * D), jnp.float32),
        in_specs=[
            pl.BlockSpec((ROW_BLK, D), lambda i: (i, 0)),
            pl.BlockSpec((D, NSLOT * D), lambda i: (0, 0)),
        ],
        out_specs=pl.BlockSpec((ROW_BLK, NSLOT * D), lambda i: (i, 0)),
        name="node_matmul",
    )(x, w)


def _stats_body(p_ref, c_ref, zr_ref, b_ref, h_ref, sums_ref, accs_ref):
    i = pl.program_id(0)
    agg = p_ref[0] + p_ref[1]
    cnt = jnp.maximum(c_ref[0, :, 0:1] + c_ref[1, :, 0:1], 1.0)
    h = agg * (1.0 / cnt) + zr_ref[...] + b_ref[...]
    h_ref[...] = h

    @pl.when(i == 0)
    def _():
        accs_ref[...] = jnp.zeros_like(accs_ref)

    accs_ref[0:1, :] += jnp.sum(h, axis=0, keepdims=True)
    accs_ref[1:2, :] += jnp.sum(h * h, axis=0, keepdims=True)

    @pl.when(i == N_GRID - 1)
    def _():
        sums_ref[...] = accs_ref[...]


def _stats(part, cnt, zroot, bias):
    return pl.pallas_call(
        _stats_body,
        grid=(N_GRID,),
        out_shape=[
            jax.ShapeDtypeStruct((N_NODES, D), jnp.float32),
            jax.ShapeDtypeStruct((2, D), jnp.float32),
        ],
        in_specs=[
            pl.BlockSpec((NC, ROW_BLK, D), lambda i: (0, i, 0)),
            pl.BlockSpec((NC, ROW_BLK, D), lambda i: (0, i, 0)),
            pl.BlockSpec((ROW_BLK, D), lambda i: (i, 0)),
            pl.BlockSpec((1, D), lambda i: (0, 0)),
        ],
        out_specs=[
            pl.BlockSpec((ROW_BLK, D), lambda i: (i, 0)),
            pl.BlockSpec((2, D), lambda i: (0, 0)),
        ],
        scratch_shapes=[pltpu.VMEM((2, D), jnp.float32)],
        name="agg_finalize_stats",
    )(part, cnt, zroot, bias)


def _bn_coeffs(sums_ref, g_ref, b_ref):
    inv_n = 1.0 / N_NODES
    mu = sums_ref[0:1, :] * inv_n
    var = sums_ref[1:2, :] * inv_n - mu * mu
    inv = lax.rsqrt(var + EPS)
    scale = g_ref[...] * inv
    shift = b_ref[...] - mu * scale
    return scale, shift


def _apply_body(h_ref, sums_ref, g_ref, b_ref, o_ref):
    scale, shift = _bn_coeffs(sums_ref, g_ref, b_ref)
    o_ref[...] = jnp.maximum(h_ref[...] * scale + shift, 0.0)


def _apply(h, sums, gamma, beta):
    return pl.pallas_call(
        _apply_body,
        grid=(N_GRID,),
        out_shape=jax.ShapeDtypeStruct((N_NODES, D), jnp.float32),
        in_specs=[
            pl.BlockSpec((ROW_BLK, D), lambda i: (i, 0)),
            pl.BlockSpec((2, D), lambda i: (0, 0)),
            pl.BlockSpec((1, D), lambda i: (0, 0)),
            pl.BlockSpec((1, D), lambda i: (0, 0)),
        ],
        out_specs=pl.BlockSpec((ROW_BLK, D), lambda i: (i, 0)),
        name="bn_relu",
    )(h, sums, gamma, beta)


def _apply_clf_body(h_ref, sums_ref, g_ref, b_ref, w_ref, cb_ref, o_ref):
    scale, shift = _bn_coeffs(sums_ref, g_ref, b_ref)
    act = jnp.maximum(h_ref[...] * scale + shift, 0.0)
    o_ref[...] = jnp.dot(act, w_ref[...],
                         preferred_element_type=jnp.float32) + cb_ref[...]


def _apply_clf(h, sums, gamma, beta, w_pad, cb_pad):
    grid = NUM_TX // ROW_BLK
    return pl.pallas_call(
        _apply_clf_body,
        grid=(grid,),
        out_shape=jax.ShapeDtypeStruct((NUM_TX, D), jnp.float32),
        in_specs=[
            pl.BlockSpec((ROW_BLK, D), lambda i: (i, 0)),
            pl.BlockSpec((2, D), lambda i: (0, 0)),
            pl.BlockSpec((1, D), lambda i: (0, 0)),
            pl.BlockSpec((1, D), lambda i: (0, 0)),
            pl.BlockSpec((D, D), lambda i: (0, 0)),
            pl.BlockSpec((1, D), lambda i: (0, 0)),
        ],
        out_specs=pl.BlockSpec((ROW_BLK, D), lambda i: (i, 0)),
        name="bn_relu_clf",
    )(h, sums, gamma, beta, w_pad, cb_pad)


def _layer(x, comp, basis, root, cbias, eidx, dst, cntpart):
    wcat = _wcat(comp, basis, root)
    z = _mm(x, wcat)
    ztab = z.reshape(N_NODES * NSLOT, D)
    zroot = z[:, R_REL * D:]
    part = _sc_agg(ztab, eidx, dst)
    return _stats(part, cntpart, zroot, cbias.reshape(1, D))


def kernel(x_transaction, emb_user, emb_device,
           comp0, basis0, root0, cbias0, gamma0, beta0,
           comp1, basis1, root1, cbias1, gamma1, beta1,
           clf_w, clf_b, edge_index, edge_type):
    x = jnp.concatenate([x_transaction, emb_user, emb_device], axis=0)
    pad = E_PAD - E_EDGES
    src = jnp.concatenate([edge_index[0].astype(jnp.int32),
                           jnp.zeros((pad,), jnp.int32)])
    dst = jnp.concatenate([edge_index[1].astype(jnp.int32),
                           jnp.full((pad,), N_NODES, jnp.int32)])
    et = jnp.concatenate([edge_type.astype(jnp.int32),
                          jnp.zeros((pad,), jnp.int32)])
    src = src.reshape(E_PAD // EB, EB)
    dst = dst.reshape(E_PAD // EB, EB)
    et = et.reshape(E_PAD // EB, EB)

    eidx = _eidx(src, et)
    cntpart = _sc_count(dst)
    h_pre, sums = _layer(x, comp0, basis0, root0, cbias0, eidx, dst,
                         cntpart)
    h = _apply(h_pre, sums, gamma0.reshape(1, D), beta0.reshape(1, D))
    h_pre, sums = _layer(h, comp1, basis1, root1, cbias1, eidx, dst,
                         cntpart)

    w_pad = jnp.pad(clf_w, ((0, 0), (0, D - clf_w.shape[1])))
    cb_pad = jnp.pad(clf_b, (0, D - clf_b.shape[0])).reshape(1, D)
    logits_pad = _apply_clf(h_pre, sums, gamma1.reshape(1, D),
                            beta1.reshape(1, D), w_pad, cb_pad)
    return logits_pad[:, :clf_w.shape[1]]


# speedup vs baseline: 1.5208x; 1.5208x over previous
import functools

import jax
import jax.numpy as jnp
from jax import lax
from jax.experimental import pallas as pl
from jax.experimental.pallas import tpu as pltpu
from jax.experimental.pallas import tpu_sc as plsc

N_NODES = 10000
NUM_TX = 6000
E_EDGES = 160000
R_REL = 8
B_BAS = 4
D = 128
NSLOT = R_REL + 1
EPS = 1e-5

NC = 2
NS = 16
LANES = 16
NW = NC * NS

EB = 128
E_PAD = 163840
EPW = E_PAD // NW
NBLK = EPW // EB
BLK_C0 = 72
BLK_C1 = 2 * NBLK - BLK_C0
BLK_MAX = max(BLK_C0, BLK_C1)
ACC_ROWS = 10112
STRIPE = ACC_ROWS // NS
_CHUNKS = []
_off = 0
while _off < STRIPE:
    _CHUNKS.append((_off, min(EB, STRIPE - _off)))
    _off += EB

ROW_BLK = 1000
N_GRID = N_NODES // ROW_BLK


def _fill_rows(rows, val16):
    def _f(i, carry):
        for j in range(D // LANES):
            rows[i, pl.ds(j * LANES, LANES)] = val16
        return carry
    lax.fori_loop(0, EB, _f, 0)


def _zero_acc_stripe(rows, acc, r0):
    _fill_rows(rows, jnp.zeros((LANES,), jnp.float32))
    for off, sz in _CHUNKS:
        pltpu.sync_copy(rows.at[pl.ds(0, sz)], acc.at[pl.ds(r0 + off, sz)])


def _writeback_stripe(acc, rows, part_out, c, r0):
    for off, sz in _CHUNKS:
        r = r0 + off
        pltpu.sync_copy(acc.at[pl.ds(r, sz)], rows.at[pl.ds(0, sz)])
        pltpu.sync_copy(rows.at[pl.ds(0, sz)], part_out.at[c, pl.ds(r, sz)])


def _sc_agg_body(z_hbm, idx_hbm, dst_hbm, part_out,
                 idx2d, dst2d, rows0, acc, semA):
    c = lax.axis_index("c")
    s = lax.axis_index("s")
    r0 = s * STRIPE

    @pl.when(c == 0)
    def _():
        pltpu.sync_copy(idx_hbm.at[pl.ds(s * BLK_C0, BLK_C0)],
                        idx2d.at[pl.ds(0, BLK_C0)])
        pltpu.sync_copy(dst_hbm.at[pl.ds(s * BLK_C0, BLK_C0)],
                        dst2d.at[pl.ds(0, BLK_C0)])

    if BLK_C1 > 0:
        @pl.when(c == 1)
        def _():
            pltpu.sync_copy(
                idx_hbm.at[pl.ds(NS * BLK_C0 + s * BLK_C1, BLK_C1)],
                idx2d.at[pl.ds(0, BLK_C1)])
            pltpu.sync_copy(
                dst_hbm.at[pl.ds(NS * BLK_C0 + s * BLK_C1, BLK_C1)],
                dst2d.at[pl.ds(0, BLK_C1)])

    nb = jnp.where(c == 0, BLK_C0, BLK_C1)
    _zero_acc_stripe(rows0, acc, r0)
    plsc.subcore_barrier()

    @pl.loop(0, nb)
    def _blk(b):
        pltpu.async_copy(z_hbm.at[idx2d.at[b]], rows0, semA).wait()
        pltpu.sync_copy(rows0, acc.at[dst2d.at[b]], add=True)

    plsc.subcore_barrier()
    _writeback_stripe(acc, rows0, part_out, c, r0)


def _sc_count_body(dst_hbm, part_out, dst2d, rows, acc, sem):
    c = lax.axis_index("c")
    s = lax.axis_index("s")
    wid = c * NS + s
    r0 = s * STRIPE
    pltpu.sync_copy(dst_hbm.at[pl.ds(wid * NBLK, NBLK)], dst2d)
    _zero_acc_stripe(rows, acc, r0)
    _fill_rows(rows, jnp.ones((LANES,), jnp.float32))
    plsc.subcore_barrier()

    descs = []
    for b in range(NBLK):
        descs.append(pltpu.async_copy(rows, acc.at[dst2d.at[b]], sem,
                                      add=True))
    for d in descs:
        d.wait()

    plsc.subcore_barrier()
    _writeback_stripe(acc, rows, part_out, c, r0)


@functools.cache
def _sc_agg_kernel():
    return pl.kernel(
        _sc_agg_body,
        out_type=[jax.ShapeDtypeStruct((NC, ACC_ROWS, D), jnp.float32)],
        mesh=plsc.VectorSubcoreMesh(core_axis_name="c", subcore_axis_name="s"),
        scratch_types=[
            pltpu.VMEM((BLK_MAX, EB), jnp.int32),
            pltpu.VMEM((BLK_MAX, EB), jnp.int32),
            pltpu.VMEM((EB, D), jnp.float32),
            pltpu.VMEM_SHARED((ACC_ROWS, D), jnp.float32),
            pltpu.SemaphoreType.DMA,
        ],
        name="sc_rgcn_agg",
    )


@functools.cache
def _sc_count_kernel():
    return pl.kernel(
        _sc_count_body,
        out_type=[jax.ShapeDtypeStruct((NC, ACC_ROWS, D), jnp.float32)],
        mesh=plsc.VectorSubcoreMesh(core_axis_name="c", subcore_axis_name="s"),
        scratch_types=[
            pltpu.VMEM((NBLK, EB), jnp.int32),
            pltpu.VMEM((EB, D), jnp.float32),
            pltpu.VMEM_SHARED((ACC_ROWS, D), jnp.float32),
            pltpu.SemaphoreType.DMA,
        ],
        name="sc_rgcn_count",
    )


def _sc_agg(ztab, eidx, dst):
    return _sc_agg_kernel()(ztab, eidx, dst)[0]


def _sc_count(dst):
    return _sc_count_kernel()(dst)[0]


def _eidx_body(s_ref, t_ref, o_ref):
    o_ref[...] = s_ref[...] * NSLOT + t_ref[...]


def _eidx(src, et):
    return pl.pallas_call(
        _eidx_body,
        out_shape=jax.ShapeDtypeStruct((E_PAD // EB, EB), jnp.int32),
        name="edge_gather_idx",
    )(src, et)


def _build_w(comp_ref, basis_ref, root_ref, w_ref):
    for r in range(R_REL):
        acc = comp_ref[r, 0] * basis_ref[0]
        for b in range(1, B_BAS):
            acc = acc + comp_ref[r, b] * basis_ref[b]
        w_ref[:, r * D:(r + 1) * D] = acc
    w_ref[:, R_REL * D:] = root_ref[...]


def _mm_body(comp_ref, basis_ref, root_ref, x_ref, o_ref, w_ref):
    @pl.when(pl.program_id(0) == 0)
    def _():
        _build_w(comp_ref, basis_ref, root_ref, w_ref)

    o_ref[...] = jnp.dot(x_ref[...], w_ref[...],
                         preferred_element_type=jnp.float32)


def _mm(comp, basis, root, x):
    return pl.pallas_call(
        _mm_body,
        grid=(N_GRID,),
        out_shape=jax.ShapeDtypeStruct((N_NODES, NSLOT * D), jnp.float32),
        in_specs=[
            pl.BlockSpec(memory_space=pltpu.SMEM),
            pl.BlockSpec((B_BAS, D, D), lambda i: (0, 0, 0)),
            pl.BlockSpec((D, D), lambda i: (0, 0)),
            pl.BlockSpec((ROW_BLK, D), lambda i: (i, 0)),
        ],
        out_specs=pl.BlockSpec((ROW_BLK, NSLOT * D), lambda i: (i, 0)),
        scratch_shapes=[pltpu.VMEM((D, NSLOT * D), jnp.float32)],
        name="node_matmul",
    )(comp, basis, root, x)


def _mm_fused_body(comp_ref, basis_ref, root_ref, h_ref, sums_ref, g_ref,
                   b_ref, o_ref, w_ref):
    @pl.when(pl.program_id(0) == 0)
    def _():
        _build_w(comp_ref, basis_ref, root_ref, w_ref)

    scale, shift = _bn_coeffs(sums_ref, g_ref, b_ref)
    act = jnp.maximum(h_ref[...] * scale + shift, 0.0)
    o_ref[...] = jnp.dot(act, w_ref[...], preferred_element_type=jnp.float32)


def _mm_fused(comp, basis, root, h_pre, sums, gamma, beta):
    return pl.pallas_call(
        _mm_fused_body,
        grid=(N_GRID,),
        out_shape=jax.ShapeDtypeStruct((N_NODES, NSLOT * D), jnp.float32),
        in_specs=[
            pl.BlockSpec(memory_space=pltpu.SMEM),
            pl.BlockSpec((B_BAS, D, D), lambda i: (0, 0, 0)),
            pl.BlockSpec((D, D), lambda i: (0, 0)),
            pl.BlockSpec((ROW_BLK, D), lambda i: (i, 0)),
            pl.BlockSpec((2, D), lambda i: (0, 0)),
            pl.BlockSpec((1, D), lambda i: (0, 0)),
            pl.BlockSpec((1, D), lambda i: (0, 0)),
        ],
        out_specs=pl.BlockSpec((ROW_BLK, NSLOT * D), lambda i: (i, 0)),
        scratch_shapes=[pltpu.VMEM((D, NSLOT * D), jnp.float32)],
        name="bn_relu_node_matmul",
    )(comp, basis, root, h_pre, sums, gamma, beta)


def _stats_body(p_ref, c_ref, zr_ref, b_ref, h_ref, sums_ref, accs_ref):
    i = pl.program_id(0)
    agg = p_ref[0] + p_ref[1]
    cnt = jnp.maximum(c_ref[0, :, 0:1] + c_ref[1, :, 0:1], 1.0)
    h = agg * (1.0 / cnt) + zr_ref[...] + b_ref[...]
    h_ref[...] = h

    @pl.when(i == 0)
    def _():
        accs_ref[...] = jnp.zeros_like(accs_ref)

    accs_ref[0:1, :] += jnp.sum(h, axis=0, keepdims=True)
    accs_ref[1:2, :] += jnp.sum(h * h, axis=0, keepdims=True)

    @pl.when(i == N_GRID - 1)
    def _():
        sums_ref[...] = accs_ref[...]


def _stats(part, cnt, zroot, bias):
    return pl.pallas_call(
        _stats_body,
        grid=(N_GRID,),
        out_shape=[
            jax.ShapeDtypeStruct((N_NODES, D), jnp.float32),
            jax.ShapeDtypeStruct((2, D), jnp.float32),
        ],
        in_specs=[
            pl.BlockSpec((NC, ROW_BLK, D), lambda i: (0, i, 0)),
            pl.BlockSpec((NC, ROW_BLK, D), lambda i: (0, i, 0)),
            pl.BlockSpec((ROW_BLK, D), lambda i: (i, 0)),
            pl.BlockSpec((1, D), lambda i: (0, 0)),
        ],
        out_specs=[
            pl.BlockSpec((ROW_BLK, D), lambda i: (i, 0)),
            pl.BlockSpec((2, D), lambda i: (0, 0)),
        ],
        scratch_shapes=[pltpu.VMEM((2, D), jnp.float32)],
        name="agg_finalize_stats",
    )(part, cnt, zroot, bias)


def _bn_coeffs(sums_ref, g_ref, b_ref):
    inv_n = 1.0 / N_NODES
    mu = sums_ref[0:1, :] * inv_n
    var = sums_ref[1:2, :] * inv_n - mu * mu
    inv = lax.rsqrt(var + EPS)
    scale = g_ref[...] * inv
    shift = b_ref[...] - mu * scale
    return scale, shift


def _apply_clf_body(h_ref, sums_ref, g_ref, b_ref, w_ref, cb_ref, o_ref):
    scale, shift = _bn_coeffs(sums_ref, g_ref, b_ref)
    act = jnp.maximum(h_ref[...] * scale + shift, 0.0)
    o_ref[...] = jnp.dot(act, w_ref[...],
                         preferred_element_type=jnp.float32) + cb_ref[...]


def _apply_clf(h, sums, gamma, beta, w_pad, cb_pad):
    grid = NUM_TX // ROW_BLK
    return pl.pallas_call(
        _apply_clf_body,
        grid=(grid,),
        out_shape=jax.ShapeDtypeStruct((NUM_TX, D), jnp.float32),
        in_specs=[
            pl.BlockSpec((ROW_BLK, D), lambda i: (i, 0)),
            pl.BlockSpec((2, D), lambda i: (0, 0)),
            pl.BlockSpec((1, D), lambda i: (0, 0)),
            pl.BlockSpec((1, D), lambda i: (0, 0)),
            pl.BlockSpec((D, D), lambda i: (0, 0)),
            pl.BlockSpec((1, D), lambda i: (0, 0)),
        ],
        out_specs=pl.BlockSpec((ROW_BLK, D), lambda i: (i, 0)),
        name="bn_relu_clf",
    )(h, sums, gamma, beta, w_pad, cb_pad)


def _layer(z, cbias, eidx, dst, cntpart):
    ztab = z.reshape(N_NODES * NSLOT, D)
    zroot = z[:, R_REL * D:]
    part = _sc_agg(ztab, eidx, dst)
    return _stats(part, cntpart, zroot, cbias.reshape(1, D))


def kernel(x_transaction, emb_user, emb_device,
           comp0, basis0, root0, cbias0, gamma0, beta0,
           comp1, basis1, root1, cbias1, gamma1, beta1,
           clf_w, clf_b, edge_index, edge_type):
    x = jnp.concatenate([x_transaction, emb_user, emb_device], axis=0)
    pad = E_PAD - E_EDGES
    src = jnp.concatenate([edge_index[0].astype(jnp.int32),
                           jnp.zeros((pad,), jnp.int32)])
    dst = jnp.concatenate([edge_index[1].astype(jnp.int32),
                           jnp.full((pad,), N_NODES, jnp.int32)])
    et = jnp.concatenate([edge_type.astype(jnp.int32),
                          jnp.zeros((pad,), jnp.int32)])
    src = src.reshape(E_PAD // EB, EB)
    dst = dst.reshape(E_PAD // EB, EB)
    et = et.reshape(E_PAD // EB, EB)

    eidx = _eidx(src, et)
    cntpart = _sc_count(dst)
    z0 = _mm(comp0, basis0, root0, x)
    h_pre, sums = _layer(z0, cbias0, eidx, dst, cntpart)
    z1 = _mm_fused(comp1, basis1, root1, h_pre, sums,
                   gamma0.reshape(1, D), beta0.reshape(1, D))
    h_pre, sums = _layer(z1, cbias1, eidx, dst, cntpart)

    w_pad = jnp.pad(clf_w, ((0, 0), (0, D - clf_w.shape[1])))
    cb_pad = jnp.pad(clf_b, (0, D - clf_b.shape[0])).reshape(1, D)
    logits_pad = _apply_clf(h_pre, sums, gamma1.reshape(1, D),
                            beta1.reshape(1, D), w_pad, cb_pad)
    return logits_pad[:, :clf_w.shape[1]]
